# baseline (speedup 1.0000x reference)
import functools

import jax
import jax.numpy as jnp
from jax import lax
from jax.experimental import pallas as pl
from jax.experimental.pallas import tpu as pltpu
from jax.experimental.pallas import tpu_sc as plsc

N = 1000000
M = 64
B = 4096
L = 50

NC = 2
NS = 16
NW = NC * NS
LANES = 16
MG = M // LANES

BAGS_W = B // NW
CB = 4
NCHUNK = BAGS_W // CB
CHUNK_ROWS = CB * L


def _embedding_bag_sum(idx2, p64, weight2):
    mesh = plsc.VectorSubcoreMesh(
        core_axis_name="c", subcore_axis_name="s",
        num_cores=NC, num_subcores=NS)

    @functools.partial(
        pl.kernel,
        out_type=jax.ShapeDtypeStruct((B, M), jnp.float32),
        mesh=mesh,
        scratch_types=[
            pltpu.VMEM((BAGS_W * L,), jnp.int32),
            pltpu.VMEM((CHUNK_ROWS, 2 * M), jnp.float32),
            pltpu.VMEM((CHUNK_ROWS, 2 * M), jnp.float32),
            pltpu.VMEM((BAGS_W, M), jnp.float32),
            pltpu.VMEM((BAGS_W * L,), jnp.int32),
            pltpu.SemaphoreType.DMA,
            pltpu.SemaphoreType.DMA,
        ],
        compiler_params=pltpu.CompilerParams(use_tc_tiling_on_sc=False,
                                             needs_layout_passes=False),
    )
    def k(idx2_hbm, p_hbm, table_hbm, out_hbm,
          idx2_v, rows0, rows1, out_v, p_v, sem0, sem1):
        wid = lax.axis_index("s") * NC + lax.axis_index("c")
        base = wid * BAGS_W * L
        pltpu.sync_copy(idx2_hbm.at[pl.ds(base, BAGS_W * L)], idx2_v)
        pltpu.sync_copy(p_hbm.at[pl.ds(base, BAGS_W * L)], p_v)
        rows = (rows0, rows1)
        sems = (sem0, sem1)
        iotas = [lax.iota(jnp.int32, LANES) + (g * LANES) for g in range(MG)]

        def start(ci, p):
            pltpu.async_copy(
                table_hbm.at[idx2_v.at[pl.ds(ci * CHUNK_ROWS, CHUNK_ROWS)]],
                rows[p], sems[p])

        start(0, 0)

        def pair(h, carry):
            for p in range(2):
                ci = h * 2 + p

                @pl.when(ci + 1 < NCHUNK)
                def _(_ci=ci, _p=p):
                    start(_ci + 1, 1 - _p)

                pltpu.make_async_copy(
                    table_hbm.at[idx2_v.at[pl.ds(ci * CHUNK_ROWS, CHUNK_ROWS)]],
                    rows[p], sems[p]).wait()
                buf = rows[p]
                for b in range(CB):
                    accs = [None] * MG
                    for r in range(L):
                        rloc = b * L + r
                        psplat = plsc.load_gather(
                            p_v, [jnp.full((LANES,), ci * CHUNK_ROWS + rloc,
                                           dtype=jnp.int32)])
                        rsplat = jnp.full((LANES,), rloc, dtype=jnp.int32)
                        for g in range(MG):
                            v = plsc.load_gather(buf, [rsplat,
                                                       psplat + iotas[g]])
                            accs[g] = v if r == 0 else accs[g] + v
                    for g in range(MG):
                        out_v[ci * CB + b, pl.ds(g * LANES, LANES)] = accs[g]
            return carry

        lax.fori_loop(0, NCHUNK // 2, pair, 0)
        pltpu.sync_copy(out_v, out_hbm.at[pl.ds(wid * BAGS_W, BAGS_W)])

    return k(idx2, p64, weight2)


def kernel(indices, offsets, valid_count_fwd, indices_bwd, offsets_bwd,
           valid_count_bwd, grad_weights, instance, weight):
    idx2 = lax.shift_right_logical(indices, 1)
    p64 = lax.shift_left(jnp.bitwise_and(indices, 1), 6)
    weight2 = weight.reshape(weight.shape[0] // 2, 2 * weight.shape[1])
    return _embedding_bag_sum(idx2, p64, weight2)


# speedup vs baseline: 1.0070x; 1.0070x over previous
import functools

import jax
import jax.numpy as jnp
from jax import lax
from jax.experimental import pallas as pl
from jax.experimental.pallas import tpu as pltpu
from jax.experimental.pallas import tpu_sc as plsc

N = 1000000
M = 64
B = 4096
L = 50

NC = 2
NS = 16
NW = NC * NS
LANES = 16
MG = M // LANES

BAGS_W = B // NW
CB = 4
NCHUNK = BAGS_W // CB
CHUNK_ROWS = CB * L


def _embedding_bag_sum(idx2, p64, weight2):
    mesh = plsc.VectorSubcoreMesh(
        core_axis_name="c", subcore_axis_name="s",
        num_cores=NC, num_subcores=NS)

    @functools.partial(
        pl.kernel,
        out_type=jax.ShapeDtypeStruct((B, M), jnp.float32),
        mesh=mesh,
        scratch_types=[
            pltpu.VMEM((BAGS_W * L,), jnp.int32),
            pltpu.VMEM((CHUNK_ROWS, 2 * M), jnp.float32),
            pltpu.VMEM((CHUNK_ROWS, 2 * M), jnp.float32),
            pltpu.VMEM((BAGS_W, M), jnp.float32),
            pltpu.VMEM((BAGS_W * L,), jnp.int32),
            pltpu.SemaphoreType.DMA,
            pltpu.SemaphoreType.DMA,
        ],
        compiler_params=pltpu.CompilerParams(use_tc_tiling_on_sc=True,
                                             needs_layout_passes=False),
    )
    def k(idx2_hbm, p_hbm, table_hbm, out_hbm,
          idx2_v, rows0, rows1, out_v, p_v, sem0, sem1):
        wid = lax.axis_index("s") * NC + lax.axis_index("c")
        base = wid * BAGS_W * L
        pltpu.sync_copy(idx2_hbm.at[pl.ds(base, BAGS_W * L)], idx2_v)
        pltpu.sync_copy(p_hbm.at[pl.ds(base, BAGS_W * L)], p_v)
        rows = (rows0, rows1)
        sems = (sem0, sem1)
        iotas = [lax.iota(jnp.int32, LANES) + (g * LANES) for g in range(MG)]

        def start(ci, p):
            pltpu.async_copy(
                table_hbm.at[idx2_v.at[pl.ds(ci * CHUNK_ROWS, CHUNK_ROWS)]],
                rows[p], sems[p])

        start(0, 0)

        def pair(h, carry):
            for p in range(2):
                ci = h * 2 + p

                @pl.when(ci + 1 < NCHUNK)
                def _(_ci=ci, _p=p):
                    start(_ci + 1, 1 - _p)

                pltpu.make_async_copy(
                    table_hbm.at[idx2_v.at[pl.ds(ci * CHUNK_ROWS, CHUNK_ROWS)]],
                    rows[p], sems[p]).wait()
                buf = rows[p]
                for b in range(CB):
                    accs = [None] * MG
                    for r in range(L):
                        rloc = b * L + r
                        psplat = plsc.load_gather(
                            p_v, [jnp.full((LANES,), ci * CHUNK_ROWS + rloc,
                                           dtype=jnp.int32)])
                        rsplat = jnp.full((LANES,), rloc, dtype=jnp.int32)
                        for g in range(MG):
                            v = plsc.load_gather(buf, [rsplat,
                                                       psplat + iotas[g]])
                            accs[g] = v if r == 0 else accs[g] + v
                    for g in range(MG):
                        out_v[ci * CB + b, pl.ds(g * LANES, LANES)] = accs[g]
            return carry

        lax.fori_loop(0, NCHUNK // 2, pair, 0)
        pltpu.sync_copy(out_v, out_hbm.at[pl.ds(wid * BAGS_W, BAGS_W)])

    return k(idx2, p64, weight2)


def kernel(indices, offsets, valid_count_fwd, indices_bwd, offsets_bwd,
           valid_count_bwd, grad_weights, instance, weight):
    idx2 = lax.shift_right_logical(indices, 1)
    p64 = lax.shift_left(jnp.bitwise_and(indices, 1), 6)
    weight2 = weight.reshape(weight.shape[0] // 2, 2 * weight.shape[1])
    return _embedding_bag_sum(idx2, p64, weight2)


# speedup vs baseline: 1.0307x; 1.0235x over previous
import functools

import jax
import jax.numpy as jnp
from jax import lax
from jax.experimental import pallas as pl
from jax.experimental.pallas import tpu as pltpu
from jax.experimental.pallas import tpu_sc as plsc

N = 1000000
M = 64
B = 4096
L = 50

NC = 2
NS = 16
NW = NC * NS
LANES = 16
MG = M // LANES

BAGS_W = B // NW
CB = 4
NCHUNK = BAGS_W // CB
CHUNK_ROWS = CB * L


def _embedding_bag_sum(idx2, p64, weight2):
    mesh = plsc.VectorSubcoreMesh(
        core_axis_name="c", subcore_axis_name="s",
        num_cores=NC, num_subcores=NS)

    @functools.partial(
        pl.kernel,
        out_type=jax.ShapeDtypeStruct((B, M), jnp.float32),
        mesh=mesh,
        scratch_types=[
            pltpu.VMEM((BAGS_W * L,), jnp.int32),
            pltpu.VMEM((CHUNK_ROWS, 2 * M), jnp.float32),
            pltpu.VMEM((CHUNK_ROWS, 2 * M), jnp.float32),
            pltpu.VMEM((BAGS_W, M), jnp.float32),
            pltpu.VMEM((BAGS_W * L,), jnp.int32),
            pltpu.SemaphoreType.DMA,
            pltpu.SemaphoreType.DMA,
        ],
        compiler_params=pltpu.CompilerParams(use_tc_tiling_on_sc=True,
                                             needs_layout_passes=False),
    )
    def k(idx2_hbm, p_hbm, table_hbm, out_hbm,
          idx2_v, rows0, rows1, out_v, p_v, sem0, sem1):
        wid = lax.axis_index("s") * NC + lax.axis_index("c")
        base = wid * BAGS_W * L
        pltpu.sync_copy(idx2_hbm.at[pl.ds(base, BAGS_W * L)], idx2_v)
        pltpu.sync_copy(p_hbm.at[pl.ds(base, BAGS_W * L)], p_v)
        rows = (rows0, rows1)
        sems = (sem0, sem1)
        iotas = [lax.iota(jnp.int32, LANES) + (g * LANES) for g in range(MG)]

        def start(ci, p):
            pltpu.async_copy(
                table_hbm.at[idx2_v.at[pl.ds(ci * CHUNK_ROWS, CHUNK_ROWS)]],
                rows[p], sems[p])

        start(0, 0)

        def pair(h, carry):
            for p in range(2):
                ci = h * 2 + p

                @pl.when(ci + 1 < NCHUNK)
                def _(_ci=ci, _p=p):
                    start(_ci + 1, 1 - _p)

                pltpu.make_async_copy(
                    table_hbm.at[idx2_v.at[pl.ds(ci * CHUNK_ROWS, CHUNK_ROWS)]],
                    rows[p], sems[p]).wait()
                buf = rows[p]
                for b in range(CB):
                    accs = [None] * MG
                    for r in range(L):
                        rloc = b * L + r
                        psplat = plsc.load_gather(
                            p_v, [jnp.full((LANES,), ci * CHUNK_ROWS + rloc,
                                           dtype=jnp.int32)])
                        rsplat = jnp.full((LANES,), rloc, dtype=jnp.int32)
                        for g in range(MG):
                            v = plsc.load_gather(buf, [rsplat,
                                                       psplat + iotas[g]])
                            accs[g] = v if r == 0 else accs[g] + v
                    for g in range(MG):
                        out_v[ci * CB + b, pl.ds(g * LANES, LANES)] = accs[g]
            return carry

        lax.fori_loop(0, NCHUNK // 2, pair, 0)
        pltpu.sync_copy(out_v, out_hbm.at[pl.ds(wid * BAGS_W, BAGS_W)])

    return k(idx2, p64, weight2)


_RB = 5000


def _repack(weight):
    n = weight.shape[0]
    n2b = n // 2 // _RB

    def body(lo_ref, hi_ref, o_ref):
        o_ref[...] = jnp.concatenate([lo_ref[...], hi_ref[...]], axis=1)

    return pl.pallas_call(
        body,
        grid=(n2b,),
        in_specs=[pl.BlockSpec((_RB, M), lambda i: (i, 0)),
                  pl.BlockSpec((_RB, M), lambda i: (i + n2b, 0))],
        out_specs=pl.BlockSpec((_RB, 2 * M), lambda i: (i, 0)),
        out_shape=jax.ShapeDtypeStruct((n // 2, 2 * M), jnp.float32),
    )(weight, weight)


def kernel(indices, offsets, valid_count_fwd, indices_bwd, offsets_bwd,
           valid_count_bwd, grad_weights, instance, weight):
    n2 = weight.shape[0] // 2
    idx2 = jnp.where(indices >= n2, indices - n2, indices)
    p64 = jnp.where(indices >= n2, jnp.int32(64), jnp.int32(0))
    weight2 = _repack(weight)
    return _embedding_bag_sum(idx2, p64, weight2)


# speedup vs baseline: 1.2485x; 1.2113x over previous
import functools

import jax
import jax.numpy as jnp
from jax import lax
from jax.experimental import pallas as pl
from jax.experimental.pallas import tpu as pltpu
from jax.experimental.pallas import tpu_sc as plsc

N = 1000000
M = 64
B = 4096
L = 50

NC = 2
NS = 16
NW = NC * NS
LANES = 16
MG = M // LANES

BAGS_W = B // NW
CB = 4
NCHUNK = BAGS_W // CB
CHUNK_ROWS = CB * L


def _embedding_bag_sum(idx2, p64, weight2):
    mesh = plsc.VectorSubcoreMesh(
        core_axis_name="c", subcore_axis_name="s",
        num_cores=NC, num_subcores=NS)

    @functools.partial(
        pl.kernel,
        out_type=jax.ShapeDtypeStruct((B, M), jnp.float32),
        mesh=mesh,
        scratch_types=[
            pltpu.VMEM((BAGS_W * L,), jnp.int32),
            pltpu.VMEM((CHUNK_ROWS, 2 * M), jnp.float32),
            pltpu.VMEM((CHUNK_ROWS, 2 * M), jnp.float32),
            pltpu.VMEM((BAGS_W, M), jnp.float32),
            pltpu.VMEM((BAGS_W * L,), jnp.int32),
            pltpu.SemaphoreType.DMA,
            pltpu.SemaphoreType.DMA,
        ],
        compiler_params=pltpu.CompilerParams(use_tc_tiling_on_sc=True,
                                             needs_layout_passes=False),
    )
    def k(idx2_hbm, p_hbm, table_hbm, out_hbm,
          idx2_v, rows0, rows1, out_v, p_v, sem0, sem1):
        wid = lax.axis_index("s") * NC + lax.axis_index("c")
        base = wid * BAGS_W * L
        pltpu.sync_copy(idx2_hbm.at[pl.ds(base, BAGS_W * L)], idx2_v)
        pltpu.sync_copy(p_hbm.at[pl.ds(base, BAGS_W * L)], p_v)
        rows = (rows0, rows1)
        sems = (sem0, sem1)
        iotas = [lax.iota(jnp.int32, LANES) + (g * LANES) for g in range(MG)]

        def start(ci, p):
            pltpu.async_copy(
                table_hbm.at[idx2_v.at[pl.ds(ci * CHUNK_ROWS, CHUNK_ROWS)]],
                rows[p], sems[p])

        start(0, 0)

        def pair(h, carry):
            for p in range(2):
                ci = h * 2 + p

                @pl.when(ci + 1 < NCHUNK)
                def _(_ci=ci, _p=p):
                    start(_ci + 1, 1 - _p)

                pltpu.make_async_copy(
                    table_hbm.at[idx2_v.at[pl.ds(ci * CHUNK_ROWS, CHUNK_ROWS)]],
                    rows[p], sems[p]).wait()
                buf = rows[p]
                for b in range(CB):
                    accs = [None] * MG
                    for r in range(L):
                        rloc = b * L + r
                        psplat = plsc.load_gather(
                            p_v, [jnp.full((LANES,), ci * CHUNK_ROWS + rloc,
                                           dtype=jnp.int32)])
                        rsplat = jnp.full((LANES,), rloc, dtype=jnp.int32)
                        for g in range(MG):
                            v = plsc.load_gather(buf, [rsplat,
                                                       psplat + iotas[g]])
                            accs[g] = v if r == 0 else accs[g] + v
                    for g in range(MG):
                        out_v[ci * CB + b, pl.ds(g * LANES, LANES)] = accs[g]
            return carry

        lax.fori_loop(0, NCHUNK // 2, pair, 0)
        pltpu.sync_copy(out_v, out_hbm.at[pl.ds(wid * BAGS_W, BAGS_W)])

    return k(idx2, p64, weight2)


_RB = 5000


def _repack(weight):
    n = weight.shape[0]
    n2b = n // 2 // _RB
    w3 = weight.reshape(2, n // 2, M)

    def body(w_ref, o_ref):
        o_ref[...] = jnp.concatenate([w_ref[0], w_ref[1]], axis=1)

    return pl.pallas_call(
        body,
        grid=(n2b,),
        in_specs=[pl.BlockSpec((2, _RB, M), lambda i: (0, i, 0))],
        out_specs=pl.BlockSpec((_RB, 2 * M), lambda i: (i, 0)),
        out_shape=jax.ShapeDtypeStruct((n // 2, 2 * M), jnp.float32),
    )(w3)


def kernel(indices, offsets, valid_count_fwd, indices_bwd, offsets_bwd,
           valid_count_bwd, grad_weights, instance, weight):
    n2 = weight.shape[0] // 2
    idx2 = jnp.where(indices >= n2, indices - n2, indices)
    p64 = jnp.where(indices >= n2, jnp.int32(64), jnp.int32(0))
    weight2 = _repack(weight)
    return _embedding_bag_sum(idx2, p64, weight2)
